# Initial kernel scaffold; baseline (speedup 1.0000x reference)
import jax
import jax.numpy as jnp
from jax.experimental import pallas as pl


def kernel(dist_inputs, grid_mask2d, cln, distance_table, region_table):
    raise NotImplementedError("write your pallas kernel here")


# speedup vs baseline: 13.3752x; 13.3752x over previous
import jax
import jax.numpy as jnp
from jax.experimental import pallas as pl
from jax.experimental.pallas import tpu as pltpu

B, L, D_CLN, D_EMB = 4, 256, 128, 64
ROWS = B * L
RB = 16


def _fused_body(dist_ref, mask_ref, cln_ref, dt_ref, rt_ref, out_ref):
    r = pl.program_id(0)
    out_ref[:, :, 0:D_CLN] = cln_ref[...]

    d = dist_ref[...]
    oh_d = (d[:, :, None] == jax.lax.broadcasted_iota(jnp.int32, (RB, L, 20), 2)
            ).astype(jnp.float32)
    dis = jnp.dot(oh_d.reshape(RB * L, 20), dt_ref[...],
                  preferred_element_type=jnp.float32)
    out_ref[:, :, D_CLN:D_CLN + D_EMB] = dis.reshape(RB, L, D_EMB)

    m = mask_ref[...]
    i_loc = jax.lax.broadcasted_iota(jnp.int32, (RB, L), 0)
    i_glob = (r * RB) % L + i_loc
    j = jax.lax.broadcasted_iota(jnp.int32, (RB, L), 1)
    reg = m * (1 + (j >= i_glob).astype(jnp.int32))
    oh_r = (reg[:, :, None] == jax.lax.broadcasted_iota(jnp.int32, (RB, L, 4), 2)
            ).astype(jnp.float32)
    rt = rt_ref[...]
    rt4 = jnp.concatenate([rt, jnp.zeros((1, D_EMB), jnp.float32)], axis=0)
    regv = jnp.dot(oh_r.reshape(RB * L, 4), rt4,
                   preferred_element_type=jnp.float32)
    out_ref[:, :, D_CLN + D_EMB:] = regv.reshape(RB, L, D_EMB)


def kernel(dist_inputs, grid_mask2d, cln, distance_table, region_table):
    dist2 = dist_inputs.reshape(ROWS, L)
    mask2 = grid_mask2d.reshape(ROWS, L)
    cln2 = cln.reshape(ROWS, L, D_CLN)
    grid = (ROWS // RB,)
    out = pl.pallas_call(
        _fused_body,
        grid=grid,
        in_specs=[
            pl.BlockSpec((RB, L), lambda r: (r, 0)),
            pl.BlockSpec((RB, L), lambda r: (r, 0)),
            pl.BlockSpec((RB, L, D_CLN), lambda r: (r, 0, 0)),
            pl.BlockSpec((20, D_EMB), lambda r: (0, 0)),
            pl.BlockSpec((3, D_EMB), lambda r: (0, 0)),
        ],
        out_specs=pl.BlockSpec((RB, L, 2 * D_CLN), lambda r: (r, 0, 0)),
        out_shape=jax.ShapeDtypeStruct((ROWS, L, 2 * D_CLN), jnp.float32),
    )(dist2, mask2, cln2, distance_table, region_table)
    return out.reshape(B, L, L, 2 * D_CLN)
